# Initial kernel scaffold; baseline (speedup 1.0000x reference)
import jax
import jax.numpy as jnp
from jax.experimental import pallas as pl


def kernel(peptide_feature, edge_index, edge_attr, Wp, bp, W1, b1, W2, b2, emb_table, gamma, beta):
    raise NotImplementedError("write your pallas kernel here")


# speedup vs baseline: 3.2679x; 3.2679x over previous
import functools

import jax
import jax.numpy as jnp
from jax import lax
from jax.experimental import pallas as pl
from jax.experimental.pallas import tpu as pltpu
from jax.experimental.pallas import tpu_sc as plsc

N = 10000
E = 320000
F = 512
H = 128
NLAYERS = 3

NC = 2
NS = 16
NW = NC * NS
EPW = E // NW
C = 80
NCHUNK = EPW // C
ZR = 624
ZREM = N - NS * ZR
HV = H // 16


def _sc_agg_body(x_hbm, src_hbm, dst_hbm, tt_hbm, emb_hbm, zero_hbm, out_hbm,
                 src_v, tt_v, dst_v, buf_v, agg_sh, sem):
    c = lax.axis_index("c")
    s = lax.axis_index("s")

    zbase = pl.multiple_of(s * ZR, 8)
    pltpu.sync_copy(zero_hbm.at[pl.ds(zbase, ZR)],
                    agg_sh.at[pl.ds(zbase, ZR)])

    @pl.when(s == NS - 1)
    def _zero_rem():
        pltpu.sync_copy(zero_hbm.at[pl.ds(NS * ZR, ZREM)],
                        agg_sh.at[pl.ds(NS * ZR, ZREM)])

    plsc.subcore_barrier()

    ebase = (c * NS + s) * EPW

    def chunk_body(k, carry):
        base = ebase + k * C
        pltpu.sync_copy(tt_hbm.at[pl.ds(base, C)], tt_v)
        pltpu.sync_copy(src_hbm.at[pl.ds(base, C)], src_v)
        pltpu.sync_copy(dst_hbm.at[pl.ds(base, C)], dst_v)
        pltpu.async_copy(emb_hbm.at[tt_v], buf_v, sem).wait()
        pltpu.async_copy(x_hbm.at[src_v], buf_v, sem, add=True).wait()

        def relu_row(r, carry2):
            for j in range(HV):
                v = buf_v[r, pl.ds(j * 16, 16)]
                buf_v[r, pl.ds(j * 16, 16)] = jnp.maximum(v, 0.0)
            return carry2
        lax.fori_loop(0, C, relu_row, 0)

        pltpu.sync_copy(buf_v, agg_sh.at[dst_v], add=True)
        return carry

    lax.fori_loop(0, NCHUNK, chunk_body, 0)
    plsc.subcore_barrier()

    wbase = pl.multiple_of(s * ZR, 8)
    pltpu.sync_copy(agg_sh.at[pl.ds(wbase, ZR)],
                    out_hbm.at[c, pl.ds(wbase, ZR)])

    @pl.when(s == NS - 1)
    def _wb_rem():
        pltpu.sync_copy(agg_sh.at[pl.ds(NS * ZR, ZREM)],
                        out_hbm.at[c, pl.ds(NS * ZR, ZREM)])


_sc_agg = pl.kernel(
    _sc_agg_body,
    out_type=jax.ShapeDtypeStruct((NC, N, H), jnp.float32),
    mesh=plsc.VectorSubcoreMesh(core_axis_name="c", subcore_axis_name="s"),
    scratch_types=[
        pltpu.VMEM((C,), jnp.int32),
        pltpu.VMEM((C,), jnp.int32),
        pltpu.VMEM((C,), jnp.int32),
        pltpu.VMEM((C, H), jnp.float32),
        pltpu.VMEM_SHARED((N, H), jnp.float32),
        pltpu.SemaphoreType.DMA,
    ],
)


BR = 1000


def _proj_body(p_ref, wp_ref, bp_ref, o_ref):
    o_ref[...] = jnp.dot(p_ref[...], wp_ref[...],
                         preferred_element_type=jnp.float32) + bp_ref[...]


_proj = pl.pallas_call(
    _proj_body,
    grid=(N // BR,),
    in_specs=[
        pl.BlockSpec((BR, F), lambda i: (i, 0)),
        pl.BlockSpec((F, H), lambda i: (0, 0)),
        pl.BlockSpec((1, H), lambda i: (0, 0)),
    ],
    out_specs=pl.BlockSpec((BR, H), lambda i: (i, 0)),
    out_shape=jax.ShapeDtypeStruct((N, H), jnp.float32),
)


def _mlp_body(x_ref, agg_ref, w1_ref, b1_ref, w2_ref, b2_ref, g_ref, be_ref,
              o_ref, *, last):
    x = x_ref[...]
    h0 = x + agg_ref[0] + agg_ref[1]
    t = jnp.maximum(jnp.dot(h0, w1_ref[...],
                            preferred_element_type=jnp.float32) + b1_ref[...],
                    0.0)
    h = jnp.dot(t, w2_ref[...],
                preferred_element_type=jnp.float32) + b2_ref[...] + x
    if last:
        mu = jnp.mean(h, axis=-1, keepdims=True)
        var = jnp.mean((h - mu) ** 2, axis=-1, keepdims=True)
        h = (h - mu) * lax.rsqrt(var + 1e-5) * g_ref[...] + be_ref[...]
    o_ref[...] = h


def _make_mlp(last):
    return pl.pallas_call(
        functools.partial(_mlp_body, last=last),
        grid=(N // BR,),
        in_specs=[
            pl.BlockSpec((BR, H), lambda i: (i, 0)),
            pl.BlockSpec((NC, BR, H), lambda i: (0, i, 0)),
            pl.BlockSpec((H, H), lambda i: (0, 0)),
            pl.BlockSpec((1, H), lambda i: (0, 0)),
            pl.BlockSpec((H, H), lambda i: (0, 0)),
            pl.BlockSpec((1, H), lambda i: (0, 0)),
            pl.BlockSpec((1, H), lambda i: (0, 0)),
            pl.BlockSpec((1, H), lambda i: (0, 0)),
        ],
        out_specs=pl.BlockSpec((BR, H), lambda i: (i, 0)),
        out_shape=jax.ShapeDtypeStruct((N, H), jnp.float32),
    )


_mlp_mid = _make_mlp(False)
_mlp_last = _make_mlp(True)


def kernel(peptide_feature, edge_index, edge_attr, Wp, bp, W1, b1, W2, b2,
           emb_table, gamma, beta):
    src = edge_index[0]
    dst = edge_index[1]
    tt = edge_attr[:, 0]
    zeros = jnp.zeros((N, H), jnp.float32)
    bp2 = bp.reshape(1, H)
    g2 = gamma.reshape(1, H)
    be2 = beta.reshape(1, H)

    x = _proj(peptide_feature, Wp, bp2)
    for i in range(NLAYERS):
        agg = _sc_agg(x, src, dst, tt, emb_table, zeros)
        mlp = _mlp_last if i == NLAYERS - 1 else _mlp_mid
        x = mlp(x, agg, W1[i], b1[i].reshape(1, H), W2[i],
                b2[i].reshape(1, H), g2, be2)
    return x


# speedup vs baseline: 5.0520x; 1.5459x over previous
import functools

import jax
import jax.numpy as jnp
from jax import lax
from jax.experimental import pallas as pl
from jax.experimental.pallas import tpu as pltpu
from jax.experimental.pallas import tpu_sc as plsc

N = 10000
E = 320000
F = 512
H = 128
NLAYERS = 3

NC = 2
NS = 16
NW = NC * NS
EPW = E // NW
C = 80
NCHUNK = EPW // C
ZR = 624
ZREM = N - NS * ZR
HV = H // 16
TPAD = 104


def _sc_agg_body(x_hbm, edata_hbm, emb_hbm, zero_hbm, out_hbm,
                 idxa_v, idxb_v, bufa_v, bufb_v, agg_sh,
                 sem_ia, sem_ib, sem_ea, sem_eb, sem_xa, sem_xb):
    c = lax.axis_index("c")
    s = lax.axis_index("s")
    w = c * NS + s

    def start_idx(k, idx, sem):
        pltpu.async_copy(edata_hbm.at[w, k], idx, sem)

    def wait_idx(idx, sem):
        pltpu.make_async_copy(edata_hbm.at[0, 0], idx, sem).wait()

    def start_e(idx, buf, sem):
        pltpu.async_copy(emb_hbm.at[idx.at[0]], buf, sem)

    def start_x(idx, buf, sem):
        pltpu.async_copy(x_hbm.at[idx.at[1]], buf, sem, add=True)

    def relu_buf(buf):
        def relu_row(r, carry):
            for j in range(HV):
                v = buf[r, pl.ds(j * 16, 16)]
                buf[r, pl.ds(j * 16, 16)] = jnp.maximum(v, 0.0)
            return carry
        lax.fori_loop(0, C, relu_row, 0)

    def scat(idx, buf):
        pltpu.sync_copy(buf, agg_sh.at[idx.at[2]], add=True)

    def wait_buf(buf, sem):
        pltpu.make_async_copy(x_hbm.at[pl.ds(0, C)], buf, sem).wait()

    pltpu.sync_copy(edata_hbm.at[w, 0], idxa_v)
    start_e(idxa_v, bufa_v, sem_ea)
    start_idx(1, idxb_v, sem_ib)

    zbase = pl.multiple_of(s * ZR, 8)
    pltpu.sync_copy(zero_hbm.at[pl.ds(zbase, ZR)],
                    agg_sh.at[pl.ds(zbase, ZR)])

    @pl.when(s == NS - 1)
    def _zero_rem():
        pltpu.sync_copy(zero_hbm.at[pl.ds(NS * ZR, ZREM)],
                        agg_sh.at[pl.ds(NS * ZR, ZREM)])

    plsc.subcore_barrier()

    def body(m, carry):
        k2 = 2 * m + 2
        k3 = 2 * m + 3
        wait_buf(bufa_v, sem_ea)
        start_x(idxa_v, bufa_v, sem_xa)
        wait_idx(idxb_v, sem_ib)
        start_e(idxb_v, bufb_v, sem_eb)
        wait_buf(bufa_v, sem_xa)
        relu_buf(bufa_v)
        scat(idxa_v, bufa_v)
        start_idx(k2, idxa_v, sem_ia)
        wait_buf(bufb_v, sem_eb)
        start_x(idxb_v, bufb_v, sem_xb)
        wait_idx(idxa_v, sem_ia)
        start_e(idxa_v, bufa_v, sem_ea)
        wait_buf(bufb_v, sem_xb)
        relu_buf(bufb_v)
        scat(idxb_v, bufb_v)

        @pl.when(k3 < NCHUNK)
        def _pf():
            start_idx(k3, idxb_v, sem_ib)
        return carry

    lax.fori_loop(0, (NCHUNK - 1) // 2, body, 0)

    wait_buf(bufa_v, sem_ea)
    start_x(idxa_v, bufa_v, sem_xa)
    wait_buf(bufa_v, sem_xa)
    relu_buf(bufa_v)
    scat(idxa_v, bufa_v)

    plsc.subcore_barrier()

    wbase = pl.multiple_of(s * ZR, 8)
    pltpu.sync_copy(agg_sh.at[pl.ds(wbase, ZR)],
                    out_hbm.at[c, pl.ds(wbase, ZR)])

    @pl.when(s == NS - 1)
    def _wb_rem():
        pltpu.sync_copy(agg_sh.at[pl.ds(NS * ZR, ZREM)],
                        out_hbm.at[c, pl.ds(NS * ZR, ZREM)])


_sc_agg = pl.kernel(
    _sc_agg_body,
    out_type=jax.ShapeDtypeStruct((NC, N, H), jnp.float32),
    mesh=plsc.VectorSubcoreMesh(core_axis_name="c", subcore_axis_name="s"),
    scratch_types=[
        pltpu.VMEM((3, C), jnp.int32),
        pltpu.VMEM((3, C), jnp.int32),
        pltpu.VMEM((C, H), jnp.float32),
        pltpu.VMEM((C, H), jnp.float32),
        pltpu.VMEM_SHARED((N, H), jnp.float32),
        pltpu.SemaphoreType.DMA,
        pltpu.SemaphoreType.DMA,
        pltpu.SemaphoreType.DMA,
        pltpu.SemaphoreType.DMA,
        pltpu.SemaphoreType.DMA,
        pltpu.SemaphoreType.DMA,
    ],
)


BR = 1000


def _proj_body(p_ref, wp_ref, bp_ref, o_ref):
    o_ref[...] = jnp.dot(p_ref[...], wp_ref[...],
                         preferred_element_type=jnp.float32) + bp_ref[...]


_proj = pl.pallas_call(
    _proj_body,
    grid=(N // BR,),
    in_specs=[
        pl.BlockSpec((BR, F), lambda i: (i, 0)),
        pl.BlockSpec((F, H), lambda i: (0, 0)),
        pl.BlockSpec((1, H), lambda i: (0, 0)),
    ],
    out_specs=pl.BlockSpec((BR, H), lambda i: (i, 0)),
    out_shape=jax.ShapeDtypeStruct((N, H), jnp.float32),
)


def _mlp_body(x_ref, agg_ref, w1_ref, b1_ref, w2_ref, b2_ref, g_ref, be_ref,
              o_ref, *, last):
    x = x_ref[...]
    h0 = x + agg_ref[0] + agg_ref[1]
    t = jnp.maximum(jnp.dot(h0, w1_ref[...],
                            preferred_element_type=jnp.float32) + b1_ref[...],
                    0.0)
    h = jnp.dot(t, w2_ref[...],
                preferred_element_type=jnp.float32) + b2_ref[...] + x
    if last:
        mu = jnp.mean(h, axis=-1, keepdims=True)
        var = jnp.mean((h - mu) ** 2, axis=-1, keepdims=True)
        h = (h - mu) * lax.rsqrt(var + 1e-5) * g_ref[...] + be_ref[...]
    o_ref[...] = h


def _make_mlp(last):
    return pl.pallas_call(
        functools.partial(_mlp_body, last=last),
        grid=(N // BR,),
        in_specs=[
            pl.BlockSpec((BR, H), lambda i: (i, 0)),
            pl.BlockSpec((NC, BR, H), lambda i: (0, i, 0)),
            pl.BlockSpec((H, H), lambda i: (0, 0)),
            pl.BlockSpec((1, H), lambda i: (0, 0)),
            pl.BlockSpec((H, H), lambda i: (0, 0)),
            pl.BlockSpec((1, H), lambda i: (0, 0)),
            pl.BlockSpec((1, H), lambda i: (0, 0)),
            pl.BlockSpec((1, H), lambda i: (0, 0)),
        ],
        out_specs=pl.BlockSpec((BR, H), lambda i: (i, 0)),
        out_shape=jax.ShapeDtypeStruct((N, H), jnp.float32),
    )


_mlp_mid = _make_mlp(False)
_mlp_last = _make_mlp(True)


def kernel(peptide_feature, edge_index, edge_attr, Wp, bp, W1, b1, W2, b2,
           emb_table, gamma, beta):
    src = edge_index[0]
    dst = edge_index[1]
    tt = edge_attr[:, 0]
    edata = jnp.stack([tt, src, dst]).reshape(3, NW, NCHUNK, C)
    edata = edata.transpose(1, 2, 0, 3)
    emb_p = jnp.zeros((TPAD, H), jnp.float32).at[:100].set(emb_table)
    zeros = jnp.zeros((N, H), jnp.float32)
    bp2 = bp.reshape(1, H)
    g2 = gamma.reshape(1, H)
    be2 = beta.reshape(1, H)

    x = _proj(peptide_feature, Wp, bp2)
    for i in range(NLAYERS):
        agg = _sc_agg(x, edata, emb_p, zeros)
        mlp = _mlp_last if i == NLAYERS - 1 else _mlp_mid
        x = mlp(x, agg, W1[i], b1[i].reshape(1, H), W2[i],
                b2[i].reshape(1, H), g2, be2)
    return x


# speedup vs baseline: 6.3599x; 1.2589x over previous
import functools

import jax
import jax.numpy as jnp
from jax import lax
from jax.experimental import pallas as pl
from jax.experimental.pallas import tpu as pltpu
from jax.experimental.pallas import tpu_sc as plsc

N = 10000
E = 320000
F = 512
H = 128
NLAYERS = 3

NC = 2
NS = 16
NW = NC * NS
EPW = E // NW
C = 80
NCHUNK = EPW // C
ZR = 624
ZREM = N - NS * ZR
HV = H // 16
TPAD = 104


def _sc_agg_body(x_hbm, edata_hbm, emb_hbm, zero_hbm, out_hbm,
                 idxa_v, idxb_v, bufa_v, bufb_v, emb_sh, agg_sh,
                 sem_ia, sem_ib, sem_ea, sem_eb, sem_xa, sem_xb):
    c = lax.axis_index("c")
    s = lax.axis_index("s")
    w = c * NS + s

    def start_idx(k, idx, sem):
        pltpu.async_copy(edata_hbm.at[w, k], idx, sem)

    def wait_idx(idx, sem):
        pltpu.make_async_copy(edata_hbm.at[0, 0], idx, sem).wait()

    def start_e(idx, buf, sem):
        pltpu.async_copy(emb_sh.at[idx.at[0]], buf, sem)

    def start_x(idx, buf, sem):
        pltpu.async_copy(x_hbm.at[idx.at[1]], buf, sem, add=True)

    def relu_buf(buf):
        def relu_row(r, carry):
            for j in range(HV):
                v = buf[r, pl.ds(j * 16, 16)]
                buf[r, pl.ds(j * 16, 16)] = jnp.maximum(v, 0.0)
            return carry
        lax.fori_loop(0, C, relu_row, 0)

    def scat(idx, buf):
        pltpu.sync_copy(buf, agg_sh.at[idx.at[2]], add=True)

    def wait_buf(buf, sem):
        pltpu.make_async_copy(x_hbm.at[pl.ds(0, C)], buf, sem).wait()

    @pl.when(s == 0)
    def _load_emb():
        pltpu.sync_copy(emb_hbm, emb_sh)

    pltpu.sync_copy(edata_hbm.at[w, 0], idxa_v)
    start_idx(1, idxb_v, sem_ib)

    zbase = pl.multiple_of(s * ZR, 8)
    pltpu.sync_copy(zero_hbm.at[pl.ds(zbase, ZR)],
                    agg_sh.at[pl.ds(zbase, ZR)])

    @pl.when(s == NS - 1)
    def _zero_rem():
        pltpu.sync_copy(zero_hbm.at[pl.ds(NS * ZR, ZREM)],
                        agg_sh.at[pl.ds(NS * ZR, ZREM)])

    plsc.subcore_barrier()
    start_e(idxa_v, bufa_v, sem_ea)

    def body(m, carry):
        k2 = 2 * m + 2
        k3 = 2 * m + 3
        wait_buf(bufa_v, sem_ea)
        start_x(idxa_v, bufa_v, sem_xa)
        wait_idx(idxb_v, sem_ib)
        start_e(idxb_v, bufb_v, sem_eb)
        wait_buf(bufa_v, sem_xa)
        relu_buf(bufa_v)
        scat(idxa_v, bufa_v)
        start_idx(k2, idxa_v, sem_ia)
        wait_buf(bufb_v, sem_eb)
        start_x(idxb_v, bufb_v, sem_xb)
        wait_idx(idxa_v, sem_ia)
        start_e(idxa_v, bufa_v, sem_ea)
        wait_buf(bufb_v, sem_xb)
        relu_buf(bufb_v)
        scat(idxb_v, bufb_v)

        @pl.when(k3 < NCHUNK)
        def _pf():
            start_idx(k3, idxb_v, sem_ib)
        return carry

    lax.fori_loop(0, (NCHUNK - 1) // 2, body, 0)

    wait_buf(bufa_v, sem_ea)
    start_x(idxa_v, bufa_v, sem_xa)
    wait_buf(bufa_v, sem_xa)
    relu_buf(bufa_v)
    scat(idxa_v, bufa_v)

    plsc.subcore_barrier()

    wbase = pl.multiple_of(s * ZR, 8)
    pltpu.sync_copy(agg_sh.at[pl.ds(wbase, ZR)],
                    out_hbm.at[c, pl.ds(wbase, ZR)])

    @pl.when(s == NS - 1)
    def _wb_rem():
        pltpu.sync_copy(agg_sh.at[pl.ds(NS * ZR, ZREM)],
                        out_hbm.at[c, pl.ds(NS * ZR, ZREM)])


_sc_agg = pl.kernel(
    _sc_agg_body,
    out_type=jax.ShapeDtypeStruct((NC, N, H), jnp.float32),
    mesh=plsc.VectorSubcoreMesh(core_axis_name="c", subcore_axis_name="s"),
    scratch_types=[
        pltpu.VMEM((3, C), jnp.int32),
        pltpu.VMEM((3, C), jnp.int32),
        pltpu.VMEM((C, H), jnp.float32),
        pltpu.VMEM((C, H), jnp.float32),
        pltpu.VMEM_SHARED((TPAD, H), jnp.float32),
        pltpu.VMEM_SHARED((N, H), jnp.float32),
        pltpu.SemaphoreType.DMA,
        pltpu.SemaphoreType.DMA,
        pltpu.SemaphoreType.DMA,
        pltpu.SemaphoreType.DMA,
        pltpu.SemaphoreType.DMA,
        pltpu.SemaphoreType.DMA,
    ],
)


BR = 1000


def _proj_body(p_ref, wp_ref, bp_ref, o_ref):
    o_ref[...] = jnp.dot(p_ref[...], wp_ref[...],
                         preferred_element_type=jnp.float32) + bp_ref[...]


_proj = pl.pallas_call(
    _proj_body,
    grid=(N // BR,),
    in_specs=[
        pl.BlockSpec((BR, F), lambda i: (i, 0)),
        pl.BlockSpec((F, H), lambda i: (0, 0)),
        pl.BlockSpec((1, H), lambda i: (0, 0)),
    ],
    out_specs=pl.BlockSpec((BR, H), lambda i: (i, 0)),
    out_shape=jax.ShapeDtypeStruct((N, H), jnp.float32),
)


def _mlp_body(x_ref, agg_ref, w1_ref, b1_ref, w2_ref, b2_ref, g_ref, be_ref,
              o_ref, *, last):
    x = x_ref[...]
    h0 = x + agg_ref[0] + agg_ref[1]
    t = jnp.maximum(jnp.dot(h0, w1_ref[...],
                            preferred_element_type=jnp.float32) + b1_ref[...],
                    0.0)
    h = jnp.dot(t, w2_ref[...],
                preferred_element_type=jnp.float32) + b2_ref[...] + x
    if last:
        mu = jnp.mean(h, axis=-1, keepdims=True)
        var = jnp.mean((h - mu) ** 2, axis=-1, keepdims=True)
        h = (h - mu) * lax.rsqrt(var + 1e-5) * g_ref[...] + be_ref[...]
    o_ref[...] = h


def _make_mlp(last):
    return pl.pallas_call(
        functools.partial(_mlp_body, last=last),
        grid=(N // BR,),
        in_specs=[
            pl.BlockSpec((BR, H), lambda i: (i, 0)),
            pl.BlockSpec((NC, BR, H), lambda i: (0, i, 0)),
            pl.BlockSpec((H, H), lambda i: (0, 0)),
            pl.BlockSpec((1, H), lambda i: (0, 0)),
            pl.BlockSpec((H, H), lambda i: (0, 0)),
            pl.BlockSpec((1, H), lambda i: (0, 0)),
            pl.BlockSpec((1, H), lambda i: (0, 0)),
            pl.BlockSpec((1, H), lambda i: (0, 0)),
        ],
        out_specs=pl.BlockSpec((BR, H), lambda i: (i, 0)),
        out_shape=jax.ShapeDtypeStruct((N, H), jnp.float32),
    )


_mlp_mid = _make_mlp(False)
_mlp_last = _make_mlp(True)


def kernel(peptide_feature, edge_index, edge_attr, Wp, bp, W1, b1, W2, b2,
           emb_table, gamma, beta):
    src = edge_index[0]
    dst = edge_index[1]
    tt = edge_attr[:, 0]
    edata = jnp.stack([tt, src, dst]).reshape(3, NW, NCHUNK, C)
    edata = edata.transpose(1, 2, 0, 3)
    emb_p = jnp.zeros((TPAD, H), jnp.float32).at[:100].set(emb_table)
    zeros = jnp.zeros((N, H), jnp.float32)
    bp2 = bp.reshape(1, H)
    g2 = gamma.reshape(1, H)
    be2 = beta.reshape(1, H)

    x = _proj(peptide_feature, Wp, bp2)
    for i in range(NLAYERS):
        agg = _sc_agg(x, edata, emb_p, zeros)
        mlp = _mlp_last if i == NLAYERS - 1 else _mlp_mid
        x = mlp(x, agg, W1[i], b1[i].reshape(1, H), W2[i],
                b2[i].reshape(1, H), g2, be2)
    return x
